# Initial kernel scaffold; baseline (speedup 1.0000x reference)
import jax
import jax.numpy as jnp
from jax.experimental import pallas as pl


def kernel(z, u_event, u_neigh, time_delta, td_event, W_t_w, W_t_b, W_rec_event_w, W_rec_event_b, W_rec_neigh_w, W_rec_neigh_b, W_e2n_w, W_e2n_b, omega_w, omega_b, w_t, alpha, psi):
    raise NotImplementedError("write your pallas kernel here")


# speedup vs baseline: 3.9832x; 3.9832x over previous
import jax
import jax.numpy as jnp
from jax import lax
from jax.experimental import pallas as pl
from jax.experimental.pallas import tpu as pltpu
from jax.experimental.pallas import tpu_sc as plsc

N = 500000
D = 64
B = 4096
K = 32
S = B * (K + 1)
NC = 2
NS = 16
NW = NC * NS
SPT = S // NS
SPW = S // NW
R_G = SPW // 128
R_T = SPT // 128
R_H = R_T // NC
PAD = 128
TDMAX = 365.0

import functools


@functools.cache
def _mesh():
    return plsc.VectorSubcoreMesh(core_axis_name="c", subcore_axis_name="s",
                                  num_cores=NC, num_subcores=NS)


def _gather_body(z_hbm, idx_hbm, out_hbm, idx_v, buf_v, sem):
    c = lax.axis_index("c")
    s = lax.axis_index("s")
    wid = s * NC + c
    pltpu.sync_copy(idx_hbm.at[wid], idx_v)

    def step(r, _):
        pltpu.async_copy(z_hbm.at[idx_v.at[r]], buf_v, sem).wait()
        pltpu.sync_copy(buf_v, out_hbm.at[pl.ds(wid * SPW + r * 128, 128)])
        return 0

    lax.fori_loop(0, R_G, step, 0)


_SC_PARAMS = pltpu.CompilerParams(use_tc_tiling_on_sc=False,
                                  needs_layout_passes=False)


@functools.cache
def _gather():
    return pl.kernel(
        _gather_body,
        out_type=jax.ShapeDtypeStruct((S, D), jnp.float32),
        mesh=_mesh(),
        compiler_params=_SC_PARAMS,
        scratch_types=[
            pltpu.VMEM((R_G, 128), jnp.int32),
            pltpu.VMEM((128, D), jnp.float32),
            pltpu.SemaphoreType.DMA,
        ],
    )


BB = 256


def _dense_body(g_ref, td_ref, tde_ref, wt_ref, bt_ref, wre_ref, bre_ref,
                wrn_ref, brn_ref, we_ref, be_ref, om_ref, ob_ref, w_t_ref,
                al_ref, ps_ref, upd_ref, lam_ref):
    dn = (((1,), (0,)), ((), ()))
    td = td_ref[...]
    tf = lax.dot_general(td.reshape(BB * (K + 1), 4), wt_ref[...], dn,
                         preferred_element_type=jnp.float32)
    tf = (tf + bt_ref[...]).reshape(BB, K + 1, D)
    zb = g_ref[...]
    z_u = zb[:, 0, :]
    pre_e = lax.dot_general(z_u, wre_ref[...], dn,
                            preferred_element_type=jnp.float32)
    z_u_new = jax.nn.sigmoid(pre_e + bre_ref[...] + tf[:, 0, :])
    msg = lax.dot_general(z_u, we_ref[...], dn,
                          preferred_element_type=jnp.float32) + be_ref[...]
    zn = zb[:, 1:, :].reshape(BB * K, D)
    pn = lax.dot_general(zn, wrn_ref[...], dn,
                         preferred_element_type=jnp.float32).reshape(BB, K, D)
    z_n_new = jax.nn.sigmoid(pn + brn_ref[...] + msg[:, None, :]
                             + tf[:, 1:, :])
    upd_ref[...] = jnp.concatenate([z_u_new[:, None, :], z_n_new], axis=1)
    g = lax.dot_general(z_u_new, om_ref[...], dn,
                        preferred_element_type=jnp.float32)[:, 0] + ob_ref[0]
    g = g + al_ref[0] * jnp.exp(-w_t_ref[0] * (tde_ref[...] / TDMAX))
    g_psi = jnp.clip(g / ps_ref[0], -75.0, 75.0)
    lam_ref[...] = ps_ref[0] * jnp.log1p(jnp.exp(g_psi))


def _full(shape):
    return pl.BlockSpec(shape, lambda i: tuple(0 for _ in shape))


def _smem_full():
    return pl.BlockSpec(memory_space=pltpu.SMEM)


_dense_in_specs = [
    pl.BlockSpec((BB, K + 1, D), lambda i: (i, 0, 0)),
    pl.BlockSpec((BB, K + 1, 4), lambda i: (i, 0, 0)),
    pl.BlockSpec((BB,), lambda i: (i,)),
    _full((4, D)), _full((D,)),
    _full((D, D)), _full((D,)),
    _full((D, D)), _full((D,)),
    _full((D, D)), _full((D,)),
    _full((D, 1)),
    _smem_full(), _smem_full(), _smem_full(), _smem_full(),
]
_dense_out_specs = [
    pl.BlockSpec((BB, K + 1, D), lambda i: (i, 0, 0)),
    pl.BlockSpec((BB,), lambda i: (i,)),
]
_dense_out_shape = [
    jax.ShapeDtypeStruct((B, K + 1, D), jnp.float32),
    jax.ShapeDtypeStruct((B,), jnp.float32),
]
_dense = pl.pallas_call(
    _dense_body,
    grid=(B // BB,),
    in_specs=_dense_in_specs,
    out_specs=_dense_out_specs,
    out_shape=_dense_out_shape,
)


def _scatter_body(idx_hbm, pri_hbm, upd_hbm, z_hbm,
                  idx_v, pri_v, got_v, con_v, sidx_v,
                  slot1_v, tgt1_v, slot2_v, tgt2_v, rows_v, stage_v, cnt_v,
                  tot_s, claim_sh, counts_sh, sem):
    c = lax.axis_index("c")
    s = lax.axis_index("s")
    pltpu.sync_copy(idx_hbm.at[s], idx_v)
    pltpu.sync_copy(pri_hbm.at[s], pri_v)
    park = N + s * 8

    def initc(i, _):
        r, k = i // 8, i % 8
        con_v[r, pl.ds(k * 16, 16)] = jnp.ones((16,), jnp.int32)
        return 0

    lax.fori_loop(0, R_T * 8, initc, 0)

    def round_body(rnd, tot):
        def work_scatter():
            def mk(i, _a):
                r, k = i // 8, i % 8
                cc = con_v[r, pl.ds(k * 16, 16)]
                ii = idx_v[r, pl.ds(k * 16, 16)]
                sidx_v[r, pl.ds(k * 16, 16)] = jnp.where(cc > 0, ii, park)
                return 0

            lax.fori_loop(0, R_T * 8, mk, 0)

            def sc_(r, _a):
                pltpu.sync_copy(pri_v.at[r], claim_sh.at[sidx_v.at[r]])
                return 0

            lax.fori_loop(0, R_T, sc_, 0)

        pl.when(tot > 0)(work_scatter)
        plsc.subcore_barrier()

        def work_gather():
            def ga(r, _a):
                pltpu.sync_copy(claim_sh.at[sidx_v.at[r]], got_v.at[r])
                return 0

            lax.fori_loop(0, R_T, ga, 0)

            def upm(i, acc):
                r, k = i // 8, i % 8
                g = got_v[r, pl.ds(k * 16, 16)]
                p = pri_v[r, pl.ds(k * 16, 16)]
                cc = con_v[r, pl.ds(k * 16, 16)]
                nc = jnp.where((g < p) & (cc > 0), 1, 0).astype(jnp.int32)
                con_v[r, pl.ds(k * 16, 16)] = nc
                return acc + nc

            acc = lax.fori_loop(0, R_T * 8, upm, jnp.zeros((16,), jnp.int32))
            stage_v[...] = jnp.full((16,), jnp.sum(acc), jnp.int32)
            pltpu.sync_copy(stage_v, counts_sh.at[s])

        pl.when(tot > 0)(work_gather)
        plsc.subcore_barrier()

        def work_total():
            pltpu.sync_copy(counts_sh, cnt_v)

            def su(t, a):
                return a + cnt_v[t]

            accg = lax.fori_loop(0, NS, su, jnp.zeros((16,), jnp.int32))
            tot_s[0] = jnp.max(accg)

        pl.when(tot > 0)(work_total)
        plsc.subcore_barrier()
        return jnp.where(tot > 0, tot_s[0], 0)

    lax.fori_loop(0, 64, round_body, jnp.int32(1))

    def ga2(r, _a):
        pltpu.sync_copy(claim_sh.at[idx_v.at[r]], got_v.at[r])
        return 0

    lax.fori_loop(0, R_T, ga2, 0)

    base_slot = s * SPT

    def comp(i, n):
        r = c * R_H + i // 8
        k = i % 8
        g = got_v[r, pl.ds(k * 16, 16)]
        p = pri_v[r, pl.ds(k * 16, 16)]
        w = g == p
        slots = (base_slot + r * 128 + k * 16
                 + lax.iota(jnp.int32, 16))
        tg = idx_v[r, pl.ds(k * 16, 16)]
        plsc.store_compressed(slot1_v.at[pl.ds(n, 16)], slots, mask=w)
        plsc.store_compressed(tgt1_v.at[pl.ds(n, 16)], tg, mask=w)
        return n + jnp.sum(w.astype(jnp.int32))

    n = lax.fori_loop(0, R_H * 8, comp, jnp.int32(0))

    nsl = (n + 127) // 128
    zero16 = jnp.zeros((16,), jnp.int32)
    s0 = jnp.take_along_axis(slot1_v[pl.ds(0, 16)], zero16, axis=0)
    t0 = jnp.take_along_axis(tgt1_v[pl.ds(0, 16)], zero16, axis=0)

    def padf(i, _a):
        slot1_v[pl.ds(n + i * 16, 16)] = s0
        tgt1_v[pl.ds(n + i * 16, 16)] = t0
        return 0

    lax.fori_loop(0, (nsl * 128 - n + 15) // 16, padf, 0)

    def c2(i, _a):
        r, k = i // 8, i % 8
        slot2_v[r, pl.ds(k * 16, 16)] = slot1_v[pl.ds(r * 128 + k * 16, 16)]
        tgt2_v[r, pl.ds(k * 16, 16)] = tgt1_v[pl.ds(r * 128 + k * 16, 16)]
        return 0

    lax.fori_loop(0, nsl * 8, c2, 0)

    def vs(r, _a):
        pltpu.async_copy(upd_hbm.at[slot2_v.at[r]], rows_v, sem).wait()
        pltpu.sync_copy(rows_v, z_hbm.at[tgt2_v.at[r]])
        return 0

    lax.fori_loop(0, nsl, vs, 0)


@functools.cache
def _scatter():
    return pl.kernel(
        _scatter_body,
        out_type=(),
        mesh=_mesh(),
        compiler_params=_SC_PARAMS,
        scratch_types=[
            pltpu.VMEM((R_T, 128), jnp.int32),
            pltpu.VMEM((R_T, 128), jnp.int32),
            pltpu.VMEM((R_T, 128), jnp.int32),
            pltpu.VMEM((R_T, 128), jnp.int32),
            pltpu.VMEM((R_T, 128), jnp.int32),
            pltpu.VMEM((SPW + PAD,), jnp.int32),
            pltpu.VMEM((SPW + PAD,), jnp.int32),
            pltpu.VMEM((R_H, 128), jnp.int32),
            pltpu.VMEM((R_H, 128), jnp.int32),
            pltpu.VMEM((128, D), jnp.float32),
            pltpu.VMEM((16,), jnp.int32),
            pltpu.VMEM((NS, 16), jnp.int32),
            pltpu.SMEM((1,), jnp.int32),
            pltpu.VMEM_SHARED((N + PAD,), jnp.int32),
            pltpu.VMEM_SHARED((NS, 16), jnp.int32),
            pltpu.SemaphoreType.DMA,
        ],
    )


def kernel(z, u_event, u_neigh, time_delta, td_event, W_t_w, W_t_b,
           W_rec_event_w, W_rec_event_b, W_rec_neigh_w, W_rec_neigh_b,
           W_e2n_w, W_e2n_b, omega_w, omega_b, w_t, alpha, psi):
    idx_all = jnp.concatenate([u_event[:, None], u_neigh], axis=1).reshape(-1)
    ar_b = jnp.arange(B, dtype=jnp.int32)
    pri = jnp.concatenate(
        [ar_b[:, None],
         B + K * ar_b[:, None] + jnp.arange(K, dtype=jnp.int32)[None, :]],
        axis=1).reshape(-1)
    sd = jnp.array([50.0, 7.0, 15.0, 15.0], jnp.float32)
    W_t_scaled = W_t_w / sd[:, None]
    gathered = _gather()(z, idx_all.reshape(NW, R_G, 128))
    upd, lam = _dense(gathered.reshape(B, K + 1, D), time_delta, td_event,
                      W_t_scaled, W_t_b, W_rec_event_w, W_rec_event_b,
                      W_rec_neigh_w, W_rec_neigh_b, W_e2n_w, W_e2n_b,
                      omega_w, omega_b, w_t, alpha, psi)
    zref = jax.new_ref(z)
    _scatter()(idx_all.reshape(NS, R_T, 128), pri.reshape(NS, R_T, 128),
               upd.reshape(S, D), zref)
    return zref[...], lam


# speedup vs baseline: 4.6374x; 1.1642x over previous
import jax
import jax.numpy as jnp
from jax import lax
from jax.experimental import pallas as pl
from jax.experimental.pallas import tpu as pltpu
from jax.experimental.pallas import tpu_sc as plsc

N = 500000
D = 64
B = 4096
K = 32
S = B * (K + 1)
NC = 2
NS = 16
NW = NC * NS
SPT = S // NS
SPW = S // NW
R_G = SPW // 128
R_T = SPT // 128
R_H = R_T // NC
PAD = 128
TDMAX = 365.0

import functools


@functools.cache
def _mesh():
    return plsc.VectorSubcoreMesh(core_axis_name="c", subcore_axis_name="s",
                                  num_cores=NC, num_subcores=NS)


def _gather_body(z_hbm, idx_hbm, out_hbm, idx_v, buf_v, sem):
    c = lax.axis_index("c")
    s = lax.axis_index("s")
    wid = s * NC + c
    pltpu.sync_copy(idx_hbm.at[wid], idx_v)

    prevd = None
    for r in range(R_G):
        cur = pltpu.async_copy(z_hbm.at[idx_v.at[r]], buf_v.at[r % 2],
                               sem.at[r % 2])
        if prevd is not None:
            prevd.wait()
            pltpu.sync_copy(buf_v.at[(r - 1) % 2],
                            out_hbm.at[pl.ds(wid * SPW + (r - 1) * 128, 128)])
        prevd = cur
    prevd.wait()
    pltpu.sync_copy(buf_v.at[(R_G - 1) % 2],
                    out_hbm.at[pl.ds(wid * SPW + (R_G - 1) * 128, 128)])


_SC_PARAMS = pltpu.CompilerParams(use_tc_tiling_on_sc=False,
                                  needs_layout_passes=False)


@functools.cache
def _gather():
    return pl.kernel(
        _gather_body,
        out_type=jax.ShapeDtypeStruct((S, D), jnp.float32),
        mesh=_mesh(),
        compiler_params=_SC_PARAMS,
        scratch_types=[
            pltpu.VMEM((R_G, 128), jnp.int32),
            pltpu.VMEM((2, 128, D), jnp.float32),
            pltpu.SemaphoreType.DMA((2,)),
        ],
    )


BB = 256


def _dense_body(g_ref, td_ref, tde_ref, wt_ref, bt_ref, wre_ref, bre_ref,
                wrn_ref, brn_ref, we_ref, be_ref, om_ref, ob_ref, w_t_ref,
                al_ref, ps_ref, upd_ref, lam_ref):
    dn = (((1,), (0,)), ((), ()))
    tf = lax.dot_general(td_ref[...], wt_ref[...], (((1,), (0,)), ((), ())),
                         preferred_element_type=jnp.float32)
    tf = tf + bt_ref[...]
    zb = g_ref[...]
    z_u = zb[0]
    pre_e = lax.dot_general(z_u, wre_ref[...], dn,
                            preferred_element_type=jnp.float32)
    z_u_new = jax.nn.sigmoid(pre_e + bre_ref[...] + tf[0])
    msg = lax.dot_general(z_u, we_ref[...], dn,
                          preferred_element_type=jnp.float32) + be_ref[...]
    zn = zb[1:].reshape(K * BB, D)
    pn = lax.dot_general(zn, wrn_ref[...], dn,
                         preferred_element_type=jnp.float32).reshape(K, BB, D)
    z_n_new = jax.nn.sigmoid(pn + brn_ref[...] + msg[None, :, :] + tf[1:])
    upd_ref[...] = jnp.concatenate([z_u_new[None], z_n_new], axis=0)
    g = lax.dot_general(z_u_new, om_ref[...], dn,
                        preferred_element_type=jnp.float32)[:, 0] + ob_ref[0]
    g = g + al_ref[0] * jnp.exp(-w_t_ref[0] * (tde_ref[...] / TDMAX))
    g_psi = jnp.clip(g / ps_ref[0], -75.0, 75.0)
    lam_ref[...] = ps_ref[0] * jnp.log1p(jnp.exp(g_psi))


def _full(shape):
    return pl.BlockSpec(shape, lambda i: tuple(0 for _ in shape))


def _smem_full():
    return pl.BlockSpec(memory_space=pltpu.SMEM)


_dense_in_specs = [
    pl.BlockSpec((K + 1, BB, D), lambda i: (0, i, 0)),
    pl.BlockSpec((K + 1, 4, BB), lambda i: (0, 0, i)),
    pl.BlockSpec((BB,), lambda i: (i,)),
    _full((4, D)), _full((D,)),
    _full((D, D)), _full((D,)),
    _full((D, D)), _full((D,)),
    _full((D, D)), _full((D,)),
    _full((D, 1)),
    _smem_full(), _smem_full(), _smem_full(), _smem_full(),
]
_dense_out_specs = [
    pl.BlockSpec((K + 1, BB, D), lambda i: (0, i, 0)),
    pl.BlockSpec((BB,), lambda i: (i,)),
]
_dense_out_shape = [
    jax.ShapeDtypeStruct((K + 1, B, D), jnp.float32),
    jax.ShapeDtypeStruct((B,), jnp.float32),
]
_dense = pl.pallas_call(
    _dense_body,
    grid=(B // BB,),
    in_specs=_dense_in_specs,
    out_specs=_dense_out_specs,
    out_shape=_dense_out_shape,
)


def _scatter_body(idx_hbm, pri_hbm, upd_hbm, z_hbm,
                  idx_v, pri_v, got_v, con_v, sidx_v,
                  slot1_v, tgt1_v, slot2_v, tgt2_v, rows_v, stage_v, cnt_v,
                  tot_s, claim_sh, counts_sh, sem, csem):
    c = lax.axis_index("c")
    s = lax.axis_index("s")
    pltpu.sync_copy(idx_hbm.at[s], idx_v)
    pltpu.sync_copy(pri_hbm.at[s], pri_v)
    park = N + s * 8

    def initc(i, _):
        r, k = i // 8, i % 8
        con_v[r, pl.ds(k * 16, 16)] = jnp.ones((16,), jnp.int32)
        return 0

    lax.fori_loop(0, R_T * 8, initc, 0)

    def round_body(rnd, tot):
        def work_scatter():
            def mk(i, _a):
                r, k = i // 8, i % 8
                cc = con_v[r, pl.ds(k * 16, 16)]
                ii = idx_v[r, pl.ds(k * 16, 16)]
                sidx_v[r, pl.ds(k * 16, 16)] = jnp.where(cc > 0, ii, park)
                return 0

            lax.fori_loop(0, R_T * 8, mk, 0)

            def sc_(r, _a):
                pltpu.async_copy(pri_v.at[r], claim_sh.at[sidx_v.at[r]], csem)
                return 0

            lax.fori_loop(0, R_T, sc_, 0)

            def scd(r, _a):
                pltpu.make_async_copy(pri_v.at[0], claim_sh.at[sidx_v.at[0]],
                                      csem).wait()
                return 0

            lax.fori_loop(0, R_T, scd, 0)

        pl.when(tot > 0)(work_scatter)
        plsc.subcore_barrier()

        def work_gather():
            def ga(r, _a):
                pltpu.async_copy(claim_sh.at[sidx_v.at[r]], got_v.at[r], csem)
                return 0

            lax.fori_loop(0, R_T, ga, 0)

            def gad(r, _a):
                pltpu.make_async_copy(claim_sh.at[sidx_v.at[0]], got_v.at[0],
                                      csem).wait()
                return 0

            lax.fori_loop(0, R_T, gad, 0)

            def upm(i, acc):
                r, k = i // 8, i % 8
                g = got_v[r, pl.ds(k * 16, 16)]
                p = pri_v[r, pl.ds(k * 16, 16)]
                cc = con_v[r, pl.ds(k * 16, 16)]
                nc = jnp.where((g < p) & (cc > 0), 1, 0).astype(jnp.int32)
                con_v[r, pl.ds(k * 16, 16)] = nc
                return acc + nc

            acc = lax.fori_loop(0, R_T * 8, upm, jnp.zeros((16,), jnp.int32))
            stage_v[...] = jnp.full((16,), jnp.sum(acc), jnp.int32)
            pltpu.sync_copy(stage_v, counts_sh.at[s])

        pl.when(tot > 0)(work_gather)
        plsc.subcore_barrier()

        def work_total():
            pltpu.sync_copy(counts_sh, cnt_v)

            def su(t, a):
                return a + cnt_v[t]

            accg = lax.fori_loop(0, NS, su, jnp.zeros((16,), jnp.int32))
            tot_s[0] = jnp.max(accg)

        pl.when(tot > 0)(work_total)
        plsc.subcore_barrier()
        return jnp.where(tot > 0, tot_s[0], 0)

    lax.fori_loop(0, 64, round_body, jnp.int32(1))

    def ga2(r, _a):
        pltpu.async_copy(claim_sh.at[idx_v.at[r]], got_v.at[r], csem)
        return 0

    lax.fori_loop(0, R_T, ga2, 0)

    def ga2d(r, _a):
        pltpu.make_async_copy(claim_sh.at[idx_v.at[0]], got_v.at[0],
                              csem).wait()
        return 0

    lax.fori_loop(0, R_T, ga2d, 0)

    base_slot = s * SPT

    def comp(i, n):
        r = c * R_H + i // 8
        k = i % 8
        g = got_v[r, pl.ds(k * 16, 16)]
        p = pri_v[r, pl.ds(k * 16, 16)]
        w = g == p
        slots = (base_slot + r * 128 + k * 16
                 + lax.iota(jnp.int32, 16))
        tg = idx_v[r, pl.ds(k * 16, 16)]
        plsc.store_compressed(slot1_v.at[pl.ds(n, 16)], slots, mask=w)
        plsc.store_compressed(tgt1_v.at[pl.ds(n, 16)], tg, mask=w)
        return n + jnp.sum(w.astype(jnp.int32))

    n = lax.fori_loop(0, R_H * 8, comp, jnp.int32(0))

    nsl = (n + 127) // 128
    zero16 = jnp.zeros((16,), jnp.int32)
    s0 = jnp.take_along_axis(slot1_v[pl.ds(0, 16)], zero16, axis=0)
    t0 = jnp.take_along_axis(tgt1_v[pl.ds(0, 16)], zero16, axis=0)

    def padf(i, _a):
        slot1_v[pl.ds(n + i * 16, 16)] = s0
        tgt1_v[pl.ds(n + i * 16, 16)] = t0
        return 0

    lax.fori_loop(0, (nsl * 128 - n + 15) // 16, padf, 0)

    def c2(i, _a):
        r, k = i // 8, i % 8
        slot2_v[r, pl.ds(k * 16, 16)] = slot1_v[pl.ds(r * 128 + k * 16, 16)]
        tgt2_v[r, pl.ds(k * 16, 16)] = tgt1_v[pl.ds(r * 128 + k * 16, 16)]
        return 0

    lax.fori_loop(0, nsl * 8, c2, 0)

    def vs(r, _a):
        pltpu.async_copy(upd_hbm.at[slot2_v.at[r]], rows_v, sem).wait()
        pltpu.sync_copy(rows_v, z_hbm.at[tgt2_v.at[r]])
        return 0

    lax.fori_loop(0, nsl, vs, 0)


@functools.cache
def _scatter():
    return pl.kernel(
        _scatter_body,
        out_type=(),
        mesh=_mesh(),
        compiler_params=_SC_PARAMS,
        scratch_types=[
            pltpu.VMEM((R_T, 128), jnp.int32),
            pltpu.VMEM((R_T, 128), jnp.int32),
            pltpu.VMEM((R_T, 128), jnp.int32),
            pltpu.VMEM((R_T, 128), jnp.int32),
            pltpu.VMEM((R_T, 128), jnp.int32),
            pltpu.VMEM((SPW + PAD,), jnp.int32),
            pltpu.VMEM((SPW + PAD,), jnp.int32),
            pltpu.VMEM((R_H, 128), jnp.int32),
            pltpu.VMEM((R_H, 128), jnp.int32),
            pltpu.VMEM((128, D), jnp.float32),
            pltpu.VMEM((16,), jnp.int32),
            pltpu.VMEM((NS, 16), jnp.int32),
            pltpu.SMEM((1,), jnp.int32),
            pltpu.VMEM_SHARED((N + PAD,), jnp.int32),
            pltpu.VMEM_SHARED((NS, 16), jnp.int32),
            pltpu.SemaphoreType.DMA,
            pltpu.SemaphoreType.DMA,
        ],
    )


def kernel(z, u_event, u_neigh, time_delta, td_event, W_t_w, W_t_b,
           W_rec_event_w, W_rec_event_b, W_rec_neigh_w, W_rec_neigh_b,
           W_e2n_w, W_e2n_b, omega_w, omega_b, w_t, alpha, psi):
    idx_all = jnp.concatenate([u_event, u_neigh.T.reshape(-1)])
    ar_b = jnp.arange(B, dtype=jnp.int32)
    pri = jnp.concatenate(
        [ar_b[None, :],
         B + K * ar_b[None, :] + jnp.arange(K, dtype=jnp.int32)[:, None]],
        axis=0).reshape(-1)
    sd = jnp.array([50.0, 7.0, 15.0, 15.0], jnp.float32)
    W_t_scaled = W_t_w / sd[:, None]
    gathered = _gather()(z, idx_all.reshape(NW, R_G, 128))
    td_t = time_delta.transpose(1, 2, 0)
    upd, lam = _dense(gathered.reshape(K + 1, B, D), td_t, td_event,
                      W_t_scaled, W_t_b, W_rec_event_w, W_rec_event_b,
                      W_rec_neigh_w, W_rec_neigh_b, W_e2n_w, W_e2n_b,
                      omega_w, omega_b, w_t, alpha, psi)
    zref = jax.new_ref(z)
    _scatter()(idx_all.reshape(NS, R_T, 128), pri.reshape(NS, R_T, 128),
               upd.reshape(S, D), zref)
    return zref[...], lam


# speedup vs baseline: 5.1832x; 1.1177x over previous
import jax
import jax.numpy as jnp
from jax import lax
from jax.experimental import pallas as pl
from jax.experimental.pallas import tpu as pltpu
from jax.experimental.pallas import tpu_sc as plsc

N = 500000
D = 64
B = 4096
K = 32
S = B * (K + 1)
NC = 2
NS = 16
NW = NC * NS
SPT = S // NS
SPW = S // NW
R_G = SPW // 128
R_T = SPT // 128
R_H = R_T // NC
PAD = 128
TDMAX = 365.0

import functools


@functools.cache
def _mesh():
    return plsc.VectorSubcoreMesh(core_axis_name="c", subcore_axis_name="s",
                                  num_cores=NC, num_subcores=NS)


def _gather_body(z_hbm, idx_hbm, out_hbm, idx_v, buf_v, sem):
    c = lax.axis_index("c")
    s = lax.axis_index("s")
    wid = s * NC + c
    pltpu.sync_copy(idx_hbm.at[wid], idx_v)

    prevd = None
    for r in range(R_G):
        cur = pltpu.async_copy(z_hbm.at[idx_v.at[r]], buf_v.at[r % 2],
                               sem.at[r % 2])
        if prevd is not None:
            prevd.wait()
            pltpu.sync_copy(buf_v.at[(r - 1) % 2],
                            out_hbm.at[pl.ds(wid * SPW + (r - 1) * 128, 128)])
        prevd = cur
    prevd.wait()
    pltpu.sync_copy(buf_v.at[(R_G - 1) % 2],
                    out_hbm.at[pl.ds(wid * SPW + (R_G - 1) * 128, 128)])


_SC_PARAMS = pltpu.CompilerParams(use_tc_tiling_on_sc=False,
                                  needs_layout_passes=False)


@functools.cache
def _gather():
    return pl.kernel(
        _gather_body,
        out_type=jax.ShapeDtypeStruct((S, D), jnp.float32),
        mesh=_mesh(),
        compiler_params=_SC_PARAMS,
        scratch_types=[
            pltpu.VMEM((R_G, 128), jnp.int32),
            pltpu.VMEM((2, 128, D), jnp.float32),
            pltpu.SemaphoreType.DMA((2,)),
        ],
    )


BB = 256


BP = B // 2
BPB = 128
D2 = 2 * D


def _dense_body(g_ref, td_ref, tde_ref, wt_ref, bt_ref, wre_ref, bre_ref,
                wrn_ref, brn_ref, we_ref, be_ref, om_ref, ob_ref, w_t_ref,
                al_ref, ps_ref, upd_ref, lam_ref):
    dn = (((1,), (0,)), ((), ()))
    tf = lax.dot_general(td_ref[...], wt_ref[...], (((2,), (0,)), ((), ())),
                         preferred_element_type=jnp.float32)
    tf = tf + bt_ref[...]
    zb = g_ref[...]
    z_u = zb[0]
    pre_e = lax.dot_general(z_u, wre_ref[...], dn,
                            preferred_element_type=jnp.float32)
    z_u_new = jax.nn.sigmoid(pre_e + bre_ref[...] + tf[0])
    msg = lax.dot_general(z_u, we_ref[...], dn,
                          preferred_element_type=jnp.float32) + be_ref[...]
    zn = zb[1:].reshape(K * BPB, D2)
    pn = lax.dot_general(zn, wrn_ref[...], dn,
                         preferred_element_type=jnp.float32).reshape(K, BPB, D2)
    z_n_new = jax.nn.sigmoid(pn + brn_ref[...] + msg[None, :, :] + tf[1:])
    upd_ref[...] = jnp.concatenate([z_u_new[None], z_n_new], axis=0)
    g = lax.dot_general(z_u_new, om_ref[...], dn,
                        preferred_element_type=jnp.float32) + ob_ref[0]
    g = g + al_ref[0] * jnp.exp(-w_t_ref[0] * (tde_ref[...] / TDMAX))
    g_psi = jnp.clip(g / ps_ref[0], -75.0, 75.0)
    lam_ref[...] = ps_ref[0] * jnp.log1p(jnp.exp(g_psi))


def _full(shape):
    return pl.BlockSpec(shape, lambda i: tuple(0 for _ in shape))


def _smem_full():
    return pl.BlockSpec(memory_space=pltpu.SMEM)


_dense_in_specs = [
    pl.BlockSpec((K + 1, BPB, D2), lambda i: (0, i, 0)),
    pl.BlockSpec((K + 1, BPB, 8), lambda i: (0, i, 0)),
    pl.BlockSpec((BPB, 2), lambda i: (i, 0)),
    _full((8, D2)), _full((D2,)),
    _full((D2, D2)), _full((D2,)),
    _full((D2, D2)), _full((D2,)),
    _full((D2, D2)), _full((D2,)),
    _full((D2, 2)),
    _smem_full(), _smem_full(), _smem_full(), _smem_full(),
]
_dense_out_specs = [
    pl.BlockSpec((K + 1, BPB, D2), lambda i: (0, i, 0)),
    pl.BlockSpec((BPB, 2), lambda i: (i, 0)),
]
_dense_out_shape = [
    jax.ShapeDtypeStruct((K + 1, BP, D2), jnp.float32),
    jax.ShapeDtypeStruct((BP, 2), jnp.float32),
]
_dense = pl.pallas_call(
    _dense_body,
    grid=(B // BB,),
    in_specs=_dense_in_specs,
    out_specs=_dense_out_specs,
    out_shape=_dense_out_shape,
)


def _scatter_body(idx_hbm, pri_hbm, upd_hbm, z_hbm,
                  idx_v, pri_v, got_v, con_v, sidx_v,
                  slot1_v, tgt1_v, slot2_v, tgt2_v, rows_v, stage_v, cnt_v,
                  tot_s, claim_sh, counts_sh, sem, csem):
    c = lax.axis_index("c")
    s = lax.axis_index("s")
    pltpu.sync_copy(idx_hbm.at[s], idx_v)
    pltpu.sync_copy(pri_hbm.at[s], pri_v)
    park = N + s * 8

    def initc(i, _):
        r, k = i // 8, i % 8
        con_v[r, pl.ds(k * 16, 16)] = jnp.ones((16,), jnp.int32)
        return 0

    lax.fori_loop(0, R_T * 8, initc, 0)

    def round_body(rnd, tot):
        def work_scatter():
            def mk(i, _a):
                r, k = i // 8, i % 8
                cc = con_v[r, pl.ds(k * 16, 16)]
                ii = idx_v[r, pl.ds(k * 16, 16)]
                sidx_v[r, pl.ds(k * 16, 16)] = jnp.where(cc > 0, ii, park)
                return 0

            lax.fori_loop(0, R_T * 8, mk, 0)

            def sc_(r, _a):
                pltpu.async_copy(pri_v.at[r], claim_sh.at[sidx_v.at[r]], csem)
                return 0

            lax.fori_loop(0, R_T, sc_, 0)

            def scd(r, _a):
                pltpu.make_async_copy(pri_v.at[0], claim_sh.at[sidx_v.at[0]],
                                      csem).wait()
                return 0

            lax.fori_loop(0, R_T, scd, 0)

        pl.when(tot > 0)(work_scatter)
        plsc.subcore_barrier()

        def work_gather():
            def ga(r, _a):
                pltpu.async_copy(claim_sh.at[sidx_v.at[r]], got_v.at[r], csem)
                return 0

            lax.fori_loop(0, R_T, ga, 0)

            def gad(r, _a):
                pltpu.make_async_copy(claim_sh.at[sidx_v.at[0]], got_v.at[0],
                                      csem).wait()
                return 0

            lax.fori_loop(0, R_T, gad, 0)

            def upm(i, acc):
                r, k = i // 8, i % 8
                g = got_v[r, pl.ds(k * 16, 16)]
                p = pri_v[r, pl.ds(k * 16, 16)]
                cc = con_v[r, pl.ds(k * 16, 16)]
                nc = jnp.where((g < p) & (cc > 0), 1, 0).astype(jnp.int32)
                con_v[r, pl.ds(k * 16, 16)] = nc
                return acc + nc

            acc = lax.fori_loop(0, R_T * 8, upm, jnp.zeros((16,), jnp.int32))
            stage_v[...] = jnp.full((16,), jnp.sum(acc), jnp.int32)
            pltpu.sync_copy(stage_v, counts_sh.at[s])

        pl.when(tot > 0)(work_gather)
        plsc.subcore_barrier()

        def work_total():
            pltpu.sync_copy(counts_sh, cnt_v)

            def su(t, a):
                return a + cnt_v[t]

            accg = lax.fori_loop(0, NS, su, jnp.zeros((16,), jnp.int32))
            tot_s[0] = jnp.max(accg)

        pl.when(tot > 0)(work_total)
        plsc.subcore_barrier()
        return jnp.where(tot > 0, tot_s[0], 0)

    lax.fori_loop(0, 64, round_body, jnp.int32(1))

    def ga2(r, _a):
        pltpu.async_copy(claim_sh.at[idx_v.at[r]], got_v.at[r], csem)
        return 0

    lax.fori_loop(0, R_T, ga2, 0)

    def ga2d(r, _a):
        pltpu.make_async_copy(claim_sh.at[idx_v.at[0]], got_v.at[0],
                              csem).wait()
        return 0

    lax.fori_loop(0, R_T, ga2d, 0)

    base_slot = s * SPT

    def comp(i, n):
        r = c * R_H + i // 8
        k = i % 8
        g = got_v[r, pl.ds(k * 16, 16)]
        p = pri_v[r, pl.ds(k * 16, 16)]
        w = g == p
        slots = (base_slot + r * 128 + k * 16
                 + lax.iota(jnp.int32, 16))
        tg = idx_v[r, pl.ds(k * 16, 16)]
        plsc.store_compressed(slot1_v.at[pl.ds(n, 16)], slots, mask=w)
        plsc.store_compressed(tgt1_v.at[pl.ds(n, 16)], tg, mask=w)
        return n + jnp.sum(w.astype(jnp.int32))

    n = lax.fori_loop(0, R_H * 8, comp, jnp.int32(0))

    nsl = (n + 127) // 128
    zero16 = jnp.zeros((16,), jnp.int32)
    s0 = jnp.take_along_axis(slot1_v[pl.ds(0, 16)], zero16, axis=0)
    t0 = jnp.take_along_axis(tgt1_v[pl.ds(0, 16)], zero16, axis=0)

    def padf(i, _a):
        slot1_v[pl.ds(n + i * 16, 16)] = s0
        tgt1_v[pl.ds(n + i * 16, 16)] = t0
        return 0

    lax.fori_loop(0, (nsl * 128 - n + 15) // 16, padf, 0)

    def c2(i, _a):
        r, k = i // 8, i % 8
        slot2_v[r, pl.ds(k * 16, 16)] = slot1_v[pl.ds(r * 128 + k * 16, 16)]
        tgt2_v[r, pl.ds(k * 16, 16)] = tgt1_v[pl.ds(r * 128 + k * 16, 16)]
        return 0

    lax.fori_loop(0, nsl * 8, c2, 0)

    def vs(r, _a):
        pltpu.async_copy(upd_hbm.at[slot2_v.at[r]], rows_v, sem).wait()
        pltpu.sync_copy(rows_v, z_hbm.at[tgt2_v.at[r]])
        return 0

    lax.fori_loop(0, nsl, vs, 0)


@functools.cache
def _scatter():
    return pl.kernel(
        _scatter_body,
        out_type=(),
        mesh=_mesh(),
        compiler_params=_SC_PARAMS,
        scratch_types=[
            pltpu.VMEM((R_T, 128), jnp.int32),
            pltpu.VMEM((R_T, 128), jnp.int32),
            pltpu.VMEM((R_T, 128), jnp.int32),
            pltpu.VMEM((R_T, 128), jnp.int32),
            pltpu.VMEM((R_T, 128), jnp.int32),
            pltpu.VMEM((SPW + PAD,), jnp.int32),
            pltpu.VMEM((SPW + PAD,), jnp.int32),
            pltpu.VMEM((R_H, 128), jnp.int32),
            pltpu.VMEM((R_H, 128), jnp.int32),
            pltpu.VMEM((128, D), jnp.float32),
            pltpu.VMEM((16,), jnp.int32),
            pltpu.VMEM((NS, 16), jnp.int32),
            pltpu.SMEM((1,), jnp.int32),
            pltpu.VMEM_SHARED((N + PAD,), jnp.int32),
            pltpu.VMEM_SHARED((NS, 16), jnp.int32),
            pltpu.SemaphoreType.DMA,
            pltpu.SemaphoreType.DMA,
        ],
    )


def kernel(z, u_event, u_neigh, time_delta, td_event, W_t_w, W_t_b,
           W_rec_event_w, W_rec_event_b, W_rec_neigh_w, W_rec_neigh_b,
           W_e2n_w, W_e2n_b, omega_w, omega_b, w_t, alpha, psi):
    idx_all = jnp.concatenate([u_event, u_neigh.T.reshape(-1)])
    ar_b = jnp.arange(B, dtype=jnp.int32)
    pri = jnp.concatenate(
        [ar_b[None, :],
         B + K * ar_b[None, :] + jnp.arange(K, dtype=jnp.int32)[:, None]],
        axis=0).reshape(-1)
    sd = jnp.array([50.0, 7.0, 15.0, 15.0], jnp.float32)
    W_t_scaled = W_t_w / sd[:, None]

    def pack2(w):
        out = jnp.zeros((D2, D2), jnp.float32)
        return out.at[:D, :D].set(w).at[D:, D:].set(w)

    wt2 = jnp.zeros((8, D2), jnp.float32)
    wt2 = wt2.at[:4, :D].set(W_t_scaled).at[4:, D:].set(W_t_scaled)
    om2 = jnp.zeros((D2, 2), jnp.float32)
    om2 = om2.at[:D, 0:1].set(omega_w).at[D:, 1:2].set(omega_w)

    gathered = _gather()(z, idx_all.reshape(NW, R_G, 128))
    tdp = time_delta.transpose(1, 0, 2).reshape(K + 1, BP, 8)
    upd, lam2 = _dense(gathered.reshape(K + 1, BP, D2), tdp,
                       td_event.reshape(BP, 2),
                       wt2, jnp.concatenate([W_t_b, W_t_b]),
                       pack2(W_rec_event_w),
                       jnp.concatenate([W_rec_event_b, W_rec_event_b]),
                       pack2(W_rec_neigh_w),
                       jnp.concatenate([W_rec_neigh_b, W_rec_neigh_b]),
                       pack2(W_e2n_w), jnp.concatenate([W_e2n_b, W_e2n_b]),
                       om2, omega_b, w_t, alpha, psi)
    zref = jax.new_ref(z)
    _scatter()(idx_all.reshape(NS, R_T, 128), pri.reshape(NS, R_T, 128),
               upd.reshape(S, D), zref)
    return zref[...], lam2.reshape(B)


# speedup vs baseline: 5.2235x; 1.0078x over previous
import jax
import jax.numpy as jnp
from jax import lax
from jax.experimental import pallas as pl
from jax.experimental.pallas import tpu as pltpu
from jax.experimental.pallas import tpu_sc as plsc

N = 500000
D = 64
B = 4096
K = 32
S = B * (K + 1)
NC = 2
NS = 16
NW = NC * NS
SPT = S // NS
SPW = S // NW
R_G = SPW // 128
R_T = SPT // 128
R_H = R_T // NC
PAD = 128
TDMAX = 365.0

import functools


@functools.cache
def _mesh():
    return plsc.VectorSubcoreMesh(core_axis_name="c", subcore_axis_name="s",
                                  num_cores=NC, num_subcores=NS)


def _gather_body(z_hbm, idx_hbm, out_hbm, idx_v, buf_v, sem):
    c = lax.axis_index("c")
    s = lax.axis_index("s")
    wid = s * NC + c
    pltpu.sync_copy(idx_hbm.at[wid], idx_v)

    prevd = None
    for r in range(R_G):
        cur = pltpu.async_copy(z_hbm.at[idx_v.at[r]], buf_v.at[r % 2],
                               sem.at[r % 2])
        if prevd is not None:
            prevd.wait()
            pltpu.sync_copy(buf_v.at[(r - 1) % 2],
                            out_hbm.at[pl.ds(wid * SPW + (r - 1) * 128, 128)])
        prevd = cur
    prevd.wait()
    pltpu.sync_copy(buf_v.at[(R_G - 1) % 2],
                    out_hbm.at[pl.ds(wid * SPW + (R_G - 1) * 128, 128)])


_SC_PARAMS = pltpu.CompilerParams(use_tc_tiling_on_sc=False,
                                  needs_layout_passes=False)


@functools.cache
def _gather():
    return pl.kernel(
        _gather_body,
        out_type=jax.ShapeDtypeStruct((S, D), jnp.float32),
        mesh=_mesh(),
        compiler_params=_SC_PARAMS,
        scratch_types=[
            pltpu.VMEM((R_G, 128), jnp.int32),
            pltpu.VMEM((2, 128, D), jnp.float32),
            pltpu.SemaphoreType.DMA((2,)),
        ],
    )


BP = B // 2
BPB = 128
D2 = 2 * D


def _dense_body(g_ref, td_ref, tde_ref, wt_ref, bt_ref, wre_ref, bre_ref,
                wrn_ref, brn_ref, we_ref, be_ref, om_ref, ob_ref, w_t_ref,
                al_ref, ps_ref, upd_ref, lam_ref):
    dn = (((1,), (0,)), ((), ()))
    tf = lax.dot_general(td_ref[...], wt_ref[...], (((2,), (0,)), ((), ())),
                         preferred_element_type=jnp.float32)
    tf = tf + bt_ref[...]
    zb = g_ref[...]
    z_u = zb[0]
    pre_e = lax.dot_general(z_u, wre_ref[...], dn,
                            preferred_element_type=jnp.float32)
    z_u_new = jax.nn.sigmoid(pre_e + bre_ref[...] + tf[0])
    msg = lax.dot_general(z_u, we_ref[...], dn,
                          preferred_element_type=jnp.float32) + be_ref[...]
    zn = zb[1:].reshape(K * BPB, D2)
    pn = lax.dot_general(zn, wrn_ref[...], dn,
                         preferred_element_type=jnp.float32).reshape(K, BPB, D2)
    z_n_new = jax.nn.sigmoid(pn + brn_ref[...] + msg[None, :, :] + tf[1:])
    upd_ref[...] = jnp.concatenate([z_u_new[None], z_n_new], axis=0)
    g = lax.dot_general(z_u_new, om_ref[...], dn,
                        preferred_element_type=jnp.float32) + ob_ref[0]
    g = g + al_ref[0] * jnp.exp(-w_t_ref[0] * (tde_ref[...] / TDMAX))
    g_psi = jnp.clip(g / ps_ref[0], -75.0, 75.0)
    lam_ref[...] = ps_ref[0] * jnp.log1p(jnp.exp(g_psi))


def _full(shape):
    return pl.BlockSpec(shape, lambda i: tuple(0 for _ in shape))


def _smem_full():
    return pl.BlockSpec(memory_space=pltpu.SMEM)


_dense_in_specs = [
    pl.BlockSpec((K + 1, BPB, D2), lambda i: (0, i, 0)),
    pl.BlockSpec((K + 1, BPB, 8), lambda i: (0, i, 0)),
    pl.BlockSpec((BPB, 2), lambda i: (i, 0)),
    _full((8, D2)), _full((D2,)),
    _full((D2, D2)), _full((D2,)),
    _full((D2, D2)), _full((D2,)),
    _full((D2, D2)), _full((D2,)),
    _full((D2, 2)),
    _smem_full(), _smem_full(), _smem_full(), _smem_full(),
]
_dense_out_specs = [
    pl.BlockSpec((K + 1, BPB, D2), lambda i: (0, i, 0)),
    pl.BlockSpec((BPB, 2), lambda i: (i, 0)),
]
_dense_out_shape = [
    jax.ShapeDtypeStruct((K + 1, BP, D2), jnp.float32),
    jax.ShapeDtypeStruct((BP, 2), jnp.float32),
]
_dense = pl.pallas_call(
    _dense_body,
    grid=(BP // BPB,),
    in_specs=_dense_in_specs,
    out_specs=_dense_out_specs,
    out_shape=_dense_out_shape,
)


def _scatter_body(idx_hbm, pri_hbm, upd_hbm, z_hbm,
                  idx_v, pri_v, got_v, con_v, sidx_v,
                  slot1_v, tgt1_v, slot2_v, tgt2_v, rows_v, stage_v, cnt_v,
                  tot_s, claim_sh, counts_sh, sem, csem):
    c = lax.axis_index("c")
    s = lax.axis_index("s")
    pltpu.sync_copy(idx_hbm.at[s], idx_v)
    pltpu.sync_copy(pri_hbm.at[s], pri_v)
    park = N + s * 8

    def initc(i, _):
        r, k = i // 8, i % 8
        con_v[r, pl.ds(k * 16, 16)] = jnp.ones((16,), jnp.int32)
        return 0

    lax.fori_loop(0, R_T * 8, initc, 0)

    def round_body(rnd, tot):
        def work_scatter():
            def mk(i, _a):
                r, k = i // 8, i % 8
                cc = con_v[r, pl.ds(k * 16, 16)]
                ii = idx_v[r, pl.ds(k * 16, 16)]
                sidx_v[r, pl.ds(k * 16, 16)] = jnp.where(cc > 0, ii, park)
                return 0

            lax.fori_loop(0, R_T * 8, mk, 0)

            def sc_(r, _a):
                pltpu.async_copy(pri_v.at[r], claim_sh.at[sidx_v.at[r]], csem)
                return 0

            lax.fori_loop(0, R_T, sc_, 0)

            def scd(r, _a):
                pltpu.make_async_copy(pri_v.at[0], claim_sh.at[sidx_v.at[0]],
                                      csem).wait()
                return 0

            lax.fori_loop(0, R_T, scd, 0)

        pl.when(tot > 0)(work_scatter)
        plsc.subcore_barrier()

        def work_gather():
            def ga(r, _a):
                pltpu.async_copy(claim_sh.at[sidx_v.at[r]], got_v.at[r], csem)
                return 0

            lax.fori_loop(0, R_T, ga, 0)

            def gad(r, _a):
                pltpu.make_async_copy(claim_sh.at[sidx_v.at[0]], got_v.at[0],
                                      csem).wait()
                return 0

            lax.fori_loop(0, R_T, gad, 0)

            def upm(i, acc):
                r, k = i // 8, i % 8
                g = got_v[r, pl.ds(k * 16, 16)]
                p = pri_v[r, pl.ds(k * 16, 16)]
                cc = con_v[r, pl.ds(k * 16, 16)]
                nc = jnp.where((g < p) & (cc > 0), 1, 0).astype(jnp.int32)
                con_v[r, pl.ds(k * 16, 16)] = nc
                return acc + nc

            acc = lax.fori_loop(0, R_T * 8, upm, jnp.zeros((16,), jnp.int32))
            stage_v[...] = jnp.full((16,), jnp.sum(acc), jnp.int32)
            pltpu.sync_copy(stage_v, counts_sh.at[s])

        pl.when(tot > 0)(work_gather)
        plsc.subcore_barrier()

        def work_total():
            pltpu.sync_copy(counts_sh, cnt_v)

            def su(t, a):
                return a + cnt_v[t]

            accg = lax.fori_loop(0, NS, su, jnp.zeros((16,), jnp.int32))
            tot_s[0] = jnp.max(accg)

        pl.when(tot > 0)(work_total)
        plsc.subcore_barrier()
        return jnp.where(tot > 0, tot_s[0], 0)

    lax.fori_loop(0, 64, round_body, jnp.int32(1))

    def ga2(r, _a):
        pltpu.async_copy(claim_sh.at[idx_v.at[r]], got_v.at[r], csem)
        return 0

    lax.fori_loop(0, R_T, ga2, 0)

    def ga2d(r, _a):
        pltpu.make_async_copy(claim_sh.at[idx_v.at[0]], got_v.at[0],
                              csem).wait()
        return 0

    lax.fori_loop(0, R_T, ga2d, 0)

    base_slot = s * SPT

    def comp(i, n):
        r = c * R_H + i // 8
        k = i % 8
        g = got_v[r, pl.ds(k * 16, 16)]
        p = pri_v[r, pl.ds(k * 16, 16)]
        w = g == p
        slots = (base_slot + r * 128 + k * 16
                 + lax.iota(jnp.int32, 16))
        tg = idx_v[r, pl.ds(k * 16, 16)]
        plsc.store_compressed(slot1_v.at[pl.ds(n, 16)], slots, mask=w)
        plsc.store_compressed(tgt1_v.at[pl.ds(n, 16)], tg, mask=w)
        return n + jnp.sum(w.astype(jnp.int32))

    n = lax.fori_loop(0, R_H * 8, comp, jnp.int32(0))

    nsl = (n + 127) // 128
    zero16 = jnp.zeros((16,), jnp.int32)
    s0 = jnp.take_along_axis(slot1_v[pl.ds(0, 16)], zero16, axis=0)
    t0 = jnp.take_along_axis(tgt1_v[pl.ds(0, 16)], zero16, axis=0)

    def padf(i, _a):
        slot1_v[pl.ds(n + i * 16, 16)] = s0
        tgt1_v[pl.ds(n + i * 16, 16)] = t0
        return 0

    lax.fori_loop(0, (nsl * 128 - n + 15) // 16, padf, 0)

    def c2(i, _a):
        r, k = i // 8, i % 8
        slot2_v[r, pl.ds(k * 16, 16)] = slot1_v[pl.ds(r * 128 + k * 16, 16)]
        tgt2_v[r, pl.ds(k * 16, 16)] = tgt1_v[pl.ds(r * 128 + k * 16, 16)]
        return 0

    lax.fori_loop(0, nsl * 8, c2, 0)

    @pl.when(nsl > 0)
    def _():
        pltpu.async_copy(upd_hbm.at[slot2_v.at[0]], rows_v.at[0], sem)

    def vs(r, _a):
        @pl.when(r % 2 == 0)
        def _():
            pltpu.make_async_copy(upd_hbm.at[slot2_v.at[r]], rows_v.at[0],
                                  sem).wait()

            @pl.when(r + 1 < nsl)
            def _():
                pltpu.async_copy(upd_hbm.at[slot2_v.at[r + 1]], rows_v.at[1],
                                 sem)

            pltpu.sync_copy(rows_v.at[0], z_hbm.at[tgt2_v.at[r]])

        @pl.when(r % 2 == 1)
        def _():
            pltpu.make_async_copy(upd_hbm.at[slot2_v.at[r]], rows_v.at[1],
                                  sem).wait()

            @pl.when(r + 1 < nsl)
            def _():
                pltpu.async_copy(upd_hbm.at[slot2_v.at[r + 1]], rows_v.at[0],
                                 sem)

            pltpu.sync_copy(rows_v.at[1], z_hbm.at[tgt2_v.at[r]])

        return 0

    lax.fori_loop(0, nsl, vs, 0)


@functools.cache
def _scatter():
    return pl.kernel(
        _scatter_body,
        out_type=(),
        mesh=_mesh(),
        compiler_params=_SC_PARAMS,
        scratch_types=[
            pltpu.VMEM((R_T, 128), jnp.int32),
            pltpu.VMEM((R_T, 128), jnp.int32),
            pltpu.VMEM((R_T, 128), jnp.int32),
            pltpu.VMEM((R_T, 128), jnp.int32),
            pltpu.VMEM((R_T, 128), jnp.int32),
            pltpu.VMEM((SPW + PAD,), jnp.int32),
            pltpu.VMEM((SPW + PAD,), jnp.int32),
            pltpu.VMEM((R_H, 128), jnp.int32),
            pltpu.VMEM((R_H, 128), jnp.int32),
            pltpu.VMEM((2, 128, D), jnp.float32),
            pltpu.VMEM((16,), jnp.int32),
            pltpu.VMEM((NS, 16), jnp.int32),
            pltpu.SMEM((1,), jnp.int32),
            pltpu.VMEM_SHARED((N + PAD,), jnp.int32),
            pltpu.VMEM_SHARED((NS, 16), jnp.int32),
            pltpu.SemaphoreType.DMA,
            pltpu.SemaphoreType.DMA,
        ],
    )


def kernel(z, u_event, u_neigh, time_delta, td_event, W_t_w, W_t_b,
           W_rec_event_w, W_rec_event_b, W_rec_neigh_w, W_rec_neigh_b,
           W_e2n_w, W_e2n_b, omega_w, omega_b, w_t, alpha, psi):
    idx_all = jnp.concatenate([u_event, u_neigh.T.reshape(-1)])
    ar_b = jnp.arange(B, dtype=jnp.int32)
    pri = jnp.concatenate(
        [ar_b[None, :],
         B + K * ar_b[None, :] + jnp.arange(K, dtype=jnp.int32)[:, None]],
        axis=0).reshape(-1)
    sd = jnp.array([50.0, 7.0, 15.0, 15.0], jnp.float32)
    W_t_scaled = W_t_w / sd[:, None]

    def pack2(w):
        out = jnp.zeros((D2, D2), jnp.float32)
        return out.at[:D, :D].set(w).at[D:, D:].set(w)

    wt2 = jnp.zeros((8, D2), jnp.float32)
    wt2 = wt2.at[:4, :D].set(W_t_scaled).at[4:, D:].set(W_t_scaled)
    om2 = jnp.zeros((D2, 2), jnp.float32)
    om2 = om2.at[:D, 0:1].set(omega_w).at[D:, 1:2].set(omega_w)

    gathered = _gather()(z, idx_all.reshape(NW, R_G, 128))
    tdp = time_delta.transpose(1, 0, 2).reshape(K + 1, BP, 8)
    upd, lam2 = _dense(gathered.reshape(K + 1, BP, D2), tdp,
                       td_event.reshape(BP, 2),
                       wt2, jnp.concatenate([W_t_b, W_t_b]),
                       pack2(W_rec_event_w),
                       jnp.concatenate([W_rec_event_b, W_rec_event_b]),
                       pack2(W_rec_neigh_w),
                       jnp.concatenate([W_rec_neigh_b, W_rec_neigh_b]),
                       pack2(W_e2n_w), jnp.concatenate([W_e2n_b, W_e2n_b]),
                       om2, omega_b, w_t, alpha, psi)
    zref = jax.new_ref(z)
    _scatter()(idx_all.reshape(NS, R_T, 128), pri.reshape(NS, R_T, 128),
               upd.reshape(S, D), zref)
    return zref[...], lam2.reshape(B)
